# initial kernel scaffold (unmeasured)
import jax
import jax.numpy as jnp
from jax import lax
from jax.experimental import pallas as pl
from jax.experimental.pallas import tpu as pltpu

N_DEV = 4
M_LOC = 1024
K = 4096
N_OUT = 8192
K_BLK = K // N_DEV
BN = 512
BQ = 1024


def _a2a_body(x_ref, out_ref, xb_ref, send_sems, recv_sems):
    my = lax.axis_index("i")

    for j in range(N_DEV):
        xb_ref[j] = x_ref[j * M_LOC:(j + 1) * M_LOC, :].astype(jnp.bfloat16)

    out_ref[my] = xb_ref[my]

    for off in range(1, N_DEV):
        d = lax.rem(my + off, N_DEV)
        rdma = pltpu.make_async_remote_copy(
            src_ref=xb_ref.at[d],
            dst_ref=out_ref.at[my],
            send_sem=send_sems.at[off - 1],
            recv_sem=recv_sems.at[my],
            device_id=(d,),
            device_id_type=pl.DeviceIdType.MESH,
        )
        rdma.start()

    for off in range(1, N_DEV):
        d = lax.rem(my + off, N_DEV)
        recv = pltpu.make_async_remote_copy(
            src_ref=xb_ref.at[d],
            dst_ref=out_ref.at[d],
            send_sem=send_sems.at[0],
            recv_sem=recv_sems.at[d],
            device_id=(d,),
            device_id_type=pl.DeviceIdType.MESH,
        )
        recv.wait_recv()

    for off in range(1, N_DEV):
        send = pltpu.make_async_remote_copy(
            src_ref=xb_ref.at[0],
            dst_ref=out_ref.at[0],
            send_sem=send_sems.at[off - 1],
            recv_sem=recv_sems.at[0],
            device_id=(0,),
            device_id_type=pl.DeviceIdType.MESH,
        )
        send.wait_send()


def _a2a(x):
    return pl.pallas_call(
        _a2a_body,
        out_shape=jax.ShapeDtypeStruct((N_DEV, M_LOC, K_BLK), jnp.bfloat16),
        in_specs=[pl.BlockSpec(memory_space=pltpu.VMEM)],
        out_specs=pl.BlockSpec(memory_space=pltpu.VMEM),
        scratch_shapes=[
            pltpu.VMEM((N_DEV, M_LOC, K_BLK), jnp.bfloat16),
            pltpu.SemaphoreType.DMA((N_DEV - 1,)),
            pltpu.SemaphoreType.DMA((N_DEV,)),
        ],
        compiler_params=pltpu.CompilerParams(collective_id=0),
    )(x)


def _gemm_body(xr_ref, w_ref, y_ref, amax_ref, acc_ref):
    n = pl.program_id(0)
    acc = jnp.dot(
        xr_ref[0],
        w_ref[0:K_BLK, :].astype(jnp.bfloat16),
        preferred_element_type=jnp.float32,
    )
    for j in range(1, N_DEV):
        acc += jnp.dot(
            xr_ref[j],
            w_ref[j * K_BLK:(j + 1) * K_BLK, :].astype(jnp.bfloat16),
            preferred_element_type=jnp.float32,
        )
    y_ref[...] = acc
    m = jnp.max(jnp.abs(acc))

    @pl.when(n == 0)
    def _():
        acc_ref[0, 0] = m

    @pl.when(n != 0)
    def _():
        acc_ref[0, 0] = jnp.maximum(acc_ref[0, 0], m)

    @pl.when(n == pl.num_programs(0) - 1)
    def _():
        amax_ref[0, 0] = acc_ref[0, 0]


def _gemm(xr, w):
    grid = (N_OUT // BN,)
    return pl.pallas_call(
        _gemm_body,
        grid=grid,
        in_specs=[
            pl.BlockSpec((N_DEV, M_LOC, K_BLK), lambda n: (0, 0, 0)),
            pl.BlockSpec((K, BN), lambda n: (0, n)),
        ],
        out_specs=[
            pl.BlockSpec((M_LOC, BN), lambda n: (0, n)),
            pl.BlockSpec((1, 1), lambda n: (0, 0), memory_space=pltpu.SMEM),
        ],
        out_shape=[
            jax.ShapeDtypeStruct((M_LOC, N_OUT), jnp.float32),
            jax.ShapeDtypeStruct((1, 1), jnp.float32),
        ],
        scratch_shapes=[pltpu.SMEM((1, 1), jnp.float32)],
        compiler_params=pltpu.CompilerParams(
            dimension_semantics=("arbitrary",),
        ),
    )(xr, w)


def _amax_body(la_ref, ga_ref, comm_ref, send_sems, recv_sems):
    my = lax.axis_index("i")
    comm_ref[my] = jnp.full((8, 128), la_ref[0, 0], jnp.float32)

    for off in range(1, N_DEV):
        d = lax.rem(my + off, N_DEV)
        rdma = pltpu.make_async_remote_copy(
            src_ref=comm_ref.at[my],
            dst_ref=comm_ref.at[my],
            send_sem=send_sems.at[off - 1],
            recv_sem=recv_sems.at[my],
            device_id=(d,),
            device_id_type=pl.DeviceIdType.MESH,
        )
        rdma.start()
    for off in range(1, N_DEV):
        d = lax.rem(my + off, N_DEV)
        recv = pltpu.make_async_remote_copy(
            src_ref=comm_ref.at[d],
            dst_ref=comm_ref.at[d],
            send_sem=send_sems.at[0],
            recv_sem=recv_sems.at[d],
            device_id=(d,),
            device_id_type=pl.DeviceIdType.MESH,
        )
        recv.wait_recv()
    for off in range(1, N_DEV):
        send = pltpu.make_async_remote_copy(
            src_ref=comm_ref.at[0],
            dst_ref=comm_ref.at[0],
            send_sem=send_sems.at[off - 1],
            recv_sem=recv_sems.at[0],
            device_id=(0,),
            device_id_type=pl.DeviceIdType.MESH,
        )
        send.wait_send()

    ga_ref[0, 0] = jnp.max(comm_ref[...])


def _amax_allreduce(la):
    return pl.pallas_call(
        _amax_body,
        out_shape=jax.ShapeDtypeStruct((1, 1), jnp.float32),
        in_specs=[pl.BlockSpec(memory_space=pltpu.SMEM)],
        out_specs=pl.BlockSpec(memory_space=pltpu.SMEM),
        scratch_shapes=[
            pltpu.VMEM((N_DEV, 8, 128), jnp.float32),
            pltpu.SemaphoreType.DMA((N_DEV - 1,)),
            pltpu.SemaphoreType.DMA((N_DEV,)),
        ],
        compiler_params=pltpu.CompilerParams(collective_id=1),
    )(la)


def _quant_body(y_ref, ga_ref, out_ref):
    s = ga_ref[0, 0] / 448.0
    q = (y_ref[...] / s).astype(jnp.float8_e4m3fn)
    out_ref[...] = q.astype(jnp.float32) * s


def _quant(y, ga):
    grid = (N_OUT // BQ,)
    return pl.pallas_call(
        _quant_body,
        grid=grid,
        in_specs=[
            pl.BlockSpec((M_LOC, BQ), lambda n: (0, n)),
            pl.BlockSpec((1, 1), lambda n: (0, 0), memory_space=pltpu.SMEM),
        ],
        out_specs=pl.BlockSpec((M_LOC, BQ), lambda n: (0, n)),
        out_shape=jax.ShapeDtypeStruct((M_LOC, N_OUT), jnp.float32),
    )(y, ga)


def kernel(x, w_mat):
    xr = _a2a(x)
    y, la = _gemm(xr, w_mat)
    ga = _amax_allreduce(la)
    return _quant(y, ga)


# baseline (device time: 172945 ns/iter reference)
import jax
import jax.numpy as jnp
from jax import lax
from jax.experimental import pallas as pl
from jax.experimental.pallas import tpu as pltpu

N_DEV = 4
M_LOC = 1024
K = 4096
N_OUT = 8192
K_BLK = K // N_DEV
BN = 512
BQ = 1024


def _a2a_body(x_ref, out_ref, xb_ref, send_sems, recv_sems):
    my = lax.axis_index("i")

    for j in range(N_DEV):
        xb_ref[j] = x_ref[j * M_LOC:(j + 1) * M_LOC, :].astype(jnp.bfloat16)

    out_ref[my] = xb_ref[my]

    for off in range(1, N_DEV):
        d = lax.rem(my + off, N_DEV)
        rdma = pltpu.make_async_remote_copy(
            src_ref=xb_ref.at[d],
            dst_ref=out_ref.at[my],
            send_sem=send_sems.at[off - 1],
            recv_sem=recv_sems.at[my],
            device_id=(d,),
            device_id_type=pl.DeviceIdType.MESH,
        )
        rdma.start()

    for off in range(1, N_DEV):
        d = lax.rem(my + off, N_DEV)
        recv = pltpu.make_async_remote_copy(
            src_ref=xb_ref.at[d],
            dst_ref=out_ref.at[d],
            send_sem=send_sems.at[0],
            recv_sem=recv_sems.at[d],
            device_id=(d,),
            device_id_type=pl.DeviceIdType.MESH,
        )
        recv.wait_recv()

    for off in range(1, N_DEV):
        send = pltpu.make_async_remote_copy(
            src_ref=xb_ref.at[0],
            dst_ref=out_ref.at[0],
            send_sem=send_sems.at[off - 1],
            recv_sem=recv_sems.at[0],
            device_id=(0,),
            device_id_type=pl.DeviceIdType.MESH,
        )
        send.wait_send()


def _a2a(x):
    return pl.pallas_call(
        _a2a_body,
        out_shape=jax.ShapeDtypeStruct((N_DEV, M_LOC, K_BLK), jnp.bfloat16),
        in_specs=[pl.BlockSpec(memory_space=pltpu.VMEM)],
        out_specs=pl.BlockSpec(memory_space=pltpu.VMEM),
        scratch_shapes=[
            pltpu.VMEM((N_DEV, M_LOC, K_BLK), jnp.bfloat16),
            pltpu.SemaphoreType.DMA((N_DEV - 1,)),
            pltpu.SemaphoreType.DMA((N_DEV,)),
        ],
    )(x)


def _gemm_body(xr_ref, w_ref, y_ref, amax_ref, acc_ref):
    n = pl.program_id(0)
    acc = jnp.dot(
        xr_ref[0],
        w_ref[0:K_BLK, :].astype(jnp.bfloat16),
        preferred_element_type=jnp.float32,
    )
    for j in range(1, N_DEV):
        acc += jnp.dot(
            xr_ref[j],
            w_ref[j * K_BLK:(j + 1) * K_BLK, :].astype(jnp.bfloat16),
            preferred_element_type=jnp.float32,
        )
    y_ref[...] = acc
    m = jnp.max(jnp.abs(acc))

    @pl.when(n == 0)
    def _():
        acc_ref[0, 0] = m

    @pl.when(n != 0)
    def _():
        acc_ref[0, 0] = jnp.maximum(acc_ref[0, 0], m)

    @pl.when(n == pl.num_programs(0) - 1)
    def _():
        amax_ref[0, 0] = acc_ref[0, 0]


def _gemm(xr, w):
    grid = (N_OUT // BN,)
    return pl.pallas_call(
        _gemm_body,
        grid=grid,
        in_specs=[
            pl.BlockSpec((N_DEV, M_LOC, K_BLK), lambda n: (0, 0, 0)),
            pl.BlockSpec((K, BN), lambda n: (0, n)),
        ],
        out_specs=[
            pl.BlockSpec((M_LOC, BN), lambda n: (0, n)),
            pl.BlockSpec((1, 1), lambda n: (0, 0), memory_space=pltpu.SMEM),
        ],
        out_shape=[
            jax.ShapeDtypeStruct((M_LOC, N_OUT), jnp.float32),
            jax.ShapeDtypeStruct((1, 1), jnp.float32),
        ],
        scratch_shapes=[pltpu.SMEM((1, 1), jnp.float32)],
        compiler_params=pltpu.CompilerParams(
            dimension_semantics=("arbitrary",),
        ),
    )(xr, w)


def _amax_body(la_ref, ga_ref, comm_ref, send_sems, recv_sems):
    my = lax.axis_index("i")
    comm_ref[my] = jnp.full((8, 128), la_ref[0, 0], jnp.float32)

    for off in range(1, N_DEV):
        d = lax.rem(my + off, N_DEV)
        rdma = pltpu.make_async_remote_copy(
            src_ref=comm_ref.at[my],
            dst_ref=comm_ref.at[my],
            send_sem=send_sems.at[off - 1],
            recv_sem=recv_sems.at[my],
            device_id=(d,),
            device_id_type=pl.DeviceIdType.MESH,
        )
        rdma.start()
    for off in range(1, N_DEV):
        d = lax.rem(my + off, N_DEV)
        recv = pltpu.make_async_remote_copy(
            src_ref=comm_ref.at[d],
            dst_ref=comm_ref.at[d],
            send_sem=send_sems.at[0],
            recv_sem=recv_sems.at[d],
            device_id=(d,),
            device_id_type=pl.DeviceIdType.MESH,
        )
        recv.wait_recv()
    for off in range(1, N_DEV):
        send = pltpu.make_async_remote_copy(
            src_ref=comm_ref.at[0],
            dst_ref=comm_ref.at[0],
            send_sem=send_sems.at[off - 1],
            recv_sem=recv_sems.at[0],
            device_id=(0,),
            device_id_type=pl.DeviceIdType.MESH,
        )
        send.wait_send()

    ga_ref[0, 0] = jnp.max(comm_ref[...])


def _amax_allreduce(la):
    return pl.pallas_call(
        _amax_body,
        out_shape=jax.ShapeDtypeStruct((1, 1), jnp.float32),
        in_specs=[pl.BlockSpec(memory_space=pltpu.SMEM)],
        out_specs=pl.BlockSpec(memory_space=pltpu.SMEM),
        scratch_shapes=[
            pltpu.VMEM((N_DEV, 8, 128), jnp.float32),
            pltpu.SemaphoreType.DMA((N_DEV - 1,)),
            pltpu.SemaphoreType.DMA((N_DEV,)),
        ],
    )(la)


def _quant_body(y_ref, ga_ref, out_ref):
    s = ga_ref[0, 0] / 448.0
    q = (y_ref[...] / s).astype(jnp.float8_e4m3fn)
    out_ref[...] = q.astype(jnp.float32) * s


def _quant(y, ga):
    grid = (N_OUT // BQ,)
    return pl.pallas_call(
        _quant_body,
        grid=grid,
        in_specs=[
            pl.BlockSpec((M_LOC, BQ), lambda n: (0, n)),
            pl.BlockSpec((1, 1), lambda n: (0, 0), memory_space=pltpu.SMEM),
        ],
        out_specs=pl.BlockSpec((M_LOC, BQ), lambda n: (0, n)),
        out_shape=jax.ShapeDtypeStruct((M_LOC, N_OUT), jnp.float32),
    )(y, ga)


def kernel(x, w_mat):
    xr = _a2a(x)
    y, la = _gemm(xr, w_mat)
    ga = _amax_allreduce(la)
    return _quant(y, ga)


# device time: 171629 ns/iter; 1.0077x vs baseline; 1.0077x over previous
import jax
import jax.numpy as jnp
from jax import lax
from jax.experimental import pallas as pl
from jax.experimental.pallas import tpu as pltpu

N_DEV = 4
M_LOC = 1024
K = 4096
N_OUT = 8192
K_BLK = K // N_DEV
BN = 512
NB = N_OUT // BN

_SEND_OFFS = [3, 1, 2]
_SLOT_FOR_OFF = {3: 1, 1: 2, 2: 3}


def _fused_body(perm_ref, x_ref, w_ref, out_ref,
                chunk_ref, xb_ref, a_ref, acc_ref, axc_ref, sm_ref,
                chunk_sem, send_sems, recv_sems,
                asend_sems, arecv_sems, out_sems):
    c = pl.program_id(0)
    n = pl.program_id(1)
    my = lax.axis_index("i")

    @pl.when((c == 0) & (n == 0))
    def _():
        for idx, off in enumerate(_SEND_OFFS):
            d = lax.rem(my + off, N_DEV)
            slot = _SLOT_FOR_OFF[off]
            cp = pltpu.make_async_copy(
                x_ref.at[pl.ds(d * M_LOC, M_LOC), :], chunk_ref, chunk_sem)
            cp.start()
            cp.wait()
            xb_ref[idx] = chunk_ref[...].astype(jnp.bfloat16)
            pltpu.make_async_remote_copy(
                src_ref=xb_ref.at[idx],
                dst_ref=a_ref.at[slot],
                send_sem=send_sems.at[idx],
                recv_sem=recv_sems.at[slot],
                device_id=(d,),
                device_id_type=pl.DeviceIdType.MESH,
            ).start()
        cp = pltpu.make_async_copy(
            x_ref.at[pl.ds(my * M_LOC, M_LOC), :], chunk_ref, chunk_sem)
        cp.start()
        cp.wait()
        a_ref[0] = chunk_ref[...].astype(jnp.bfloat16)

    @pl.when((c >= 1) & (c <= 3) & (n == 0))
    def _():
        pltpu.make_async_remote_copy(
            src_ref=xb_ref.at[0],
            dst_ref=a_ref.at[c],
            send_sem=send_sems.at[0],
            recv_sem=recv_sems.at[c],
            device_id=(0,),
            device_id_type=pl.DeviceIdType.MESH,
        ).wait_recv()

    @pl.when(c <= 3)
    def _():
        wb = w_ref[...].astype(jnp.bfloat16)
        partial = jax.lax.dot_general(
            a_ref[c], wb, (((1,), (0,)), ((), ())),
            preferred_element_type=jnp.float32,
        )

        @pl.when(c == 0)
        def _():
            acc_ref[n] = partial

        @pl.when((c == 1) | (c == 2))
        def _():
            acc_ref[n] = acc_ref[n] + partial

        @pl.when(c == 3)
        def _():
            final = acc_ref[n] + partial
            acc_ref[n] = final
            m = jnp.max(jnp.abs(final))

            @pl.when(n == 0)
            def _():
                sm_ref[0] = m

            @pl.when(n > 0)
            def _():
                sm_ref[0] = jnp.maximum(sm_ref[0], m)

    @pl.when((c == 4) & (n == 0))
    def _():
        axc_ref[my] = jnp.full((8, 128), sm_ref[0], jnp.float32)
        for off in range(1, N_DEV):
            d = lax.rem(my + off, N_DEV)
            pltpu.make_async_remote_copy(
                src_ref=axc_ref.at[my],
                dst_ref=axc_ref.at[my],
                send_sem=asend_sems.at[off - 1],
                recv_sem=arecv_sems.at[my],
                device_id=(d,),
                device_id_type=pl.DeviceIdType.MESH,
            ).start()
        for off in range(1, N_DEV):
            d = lax.rem(my + off, N_DEV)
            pltpu.make_async_remote_copy(
                src_ref=axc_ref.at[d],
                dst_ref=axc_ref.at[d],
                send_sem=asend_sems.at[0],
                recv_sem=arecv_sems.at[d],
                device_id=(d,),
                device_id_type=pl.DeviceIdType.MESH,
            ).wait_recv()
        sm_ref[1] = jnp.max(axc_ref[...]) / 448.0

    @pl.when(c == 4)
    def _():
        s = sm_ref[1]
        q = (acc_ref[n] / s).astype(jnp.float8_e4m3fn)
        acc_ref[n] = q.astype(jnp.float32) * s

        @pl.when(n >= 2)
        def _():
            pltpu.make_async_copy(
                acc_ref.at[n - 2],
                out_ref.at[:, pl.ds((n - 2) * BN, BN)],
                out_sems.at[lax.rem(n, 2)],
            ).wait()

        pltpu.make_async_copy(
            acc_ref.at[n],
            out_ref.at[:, pl.ds(n * BN, BN)],
            out_sems.at[lax.rem(n, 2)],
        ).start()

    @pl.when((c == 4) & (n == NB - 1))
    def _():
        pltpu.make_async_copy(
            acc_ref.at[NB - 2], out_ref.at[:, pl.ds((NB - 2) * BN, BN)],
            out_sems.at[0]).wait()
        pltpu.make_async_copy(
            acc_ref.at[NB - 1], out_ref.at[:, pl.ds((NB - 1) * BN, BN)],
            out_sems.at[1]).wait()
        for idx in range(3):
            pltpu.make_async_remote_copy(
                src_ref=xb_ref.at[idx], dst_ref=a_ref.at[1],
                send_sem=send_sems.at[idx], recv_sem=recv_sems.at[1],
                device_id=(0,), device_id_type=pl.DeviceIdType.MESH,
            ).wait_send()
            pltpu.make_async_remote_copy(
                src_ref=axc_ref.at[0], dst_ref=axc_ref.at[0],
                send_sem=asend_sems.at[idx], recv_sem=arecv_sems.at[0],
                device_id=(0,), device_id_type=pl.DeviceIdType.MESH,
            ).wait_send()


def _w_index_map(c, n, perm_ref):
    return (perm_ref[jnp.minimum(c, 3)], jnp.where(c == 4, NB - 1, n))


def kernel(x, w_mat):
    my = lax.axis_index("i")
    perm = jnp.remainder(
        jnp.array([0, 1, 3, 2], jnp.int32) + my.astype(jnp.int32), N_DEV)

    grid_spec = pltpu.PrefetchScalarGridSpec(
        num_scalar_prefetch=1,
        grid=(N_DEV + 1, NB),
        in_specs=[
            pl.BlockSpec(memory_space=pl.ANY),
            pl.BlockSpec((K_BLK, BN), _w_index_map),
        ],
        out_specs=pl.BlockSpec(memory_space=pl.ANY),
        scratch_shapes=[
            pltpu.VMEM((M_LOC, K_BLK), jnp.float32),
            pltpu.VMEM((3, M_LOC, K_BLK), jnp.bfloat16),
            pltpu.VMEM((N_DEV, M_LOC, K_BLK), jnp.bfloat16),
            pltpu.VMEM((NB, M_LOC, BN), jnp.float32),
            pltpu.VMEM((N_DEV, 8, 128), jnp.float32),
            pltpu.SMEM((2,), jnp.float32),
            pltpu.SemaphoreType.DMA,
            pltpu.SemaphoreType.DMA((3,)),
            pltpu.SemaphoreType.DMA((N_DEV,)),
            pltpu.SemaphoreType.DMA((3,)),
            pltpu.SemaphoreType.DMA((N_DEV,)),
            pltpu.SemaphoreType.DMA((2,)),
        ],
    )
    return pl.pallas_call(
        _fused_body,
        grid_spec=grid_spec,
        out_shape=jax.ShapeDtypeStruct((M_LOC, N_OUT), jnp.float32),
        compiler_params=pltpu.CompilerParams(
            dimension_semantics=("arbitrary", "arbitrary"),
            vmem_limit_bytes=64 * 1024 * 1024,
        ),
    )(perm, x, w_mat)


# device time: 164240 ns/iter; 1.0530x vs baseline; 1.0450x over previous
import jax
import jax.numpy as jnp
from jax import lax
from jax.experimental import pallas as pl
from jax.experimental.pallas import tpu as pltpu

N_DEV = 4
M_LOC = 1024
K = 4096
N_OUT = 8192
K_BLK = K // N_DEV
BN = 512
NB = N_OUT // BN

_SEND_OFFS = [3, 1, 2]
_SLOT_FOR_OFF = {3: 1, 1: 2, 2: 3}


def _fused_body(perm_ref, x_ref, w_ref, out_ref,
                chunk_ref, xb_ref, a_ref, acc_ref, axc_ref, sm_ref,
                chunk_sem, send_sems, recv_sems,
                asend_sems, arecv_sems, out_sems):
    c = pl.program_id(0)
    n = pl.program_id(1)
    my = lax.axis_index("i")

    @pl.when((c == 0) & (n == 0))
    def _():
        for idx, off in enumerate(_SEND_OFFS):
            d = lax.rem(my + off, N_DEV)
            slot = _SLOT_FOR_OFF[off]
            cp = pltpu.make_async_copy(
                x_ref.at[pl.ds(d * M_LOC, M_LOC), :], chunk_ref, chunk_sem)
            cp.start()
            cp.wait()
            xb_ref[idx] = chunk_ref[...].astype(jnp.bfloat16)
            pltpu.make_async_remote_copy(
                src_ref=xb_ref.at[idx],
                dst_ref=a_ref.at[slot],
                send_sem=send_sems.at[idx],
                recv_sem=recv_sems.at[slot],
                device_id=(d,),
                device_id_type=pl.DeviceIdType.MESH,
            ).start()
        cp = pltpu.make_async_copy(
            x_ref.at[pl.ds(my * M_LOC, M_LOC), :], chunk_ref, chunk_sem)
        cp.start()
        cp.wait()
        a_ref[0] = chunk_ref[...].astype(jnp.bfloat16)

    @pl.when((c >= 1) & (c <= 3) & (n == 0))
    def _():
        pltpu.make_async_remote_copy(
            src_ref=xb_ref.at[0],
            dst_ref=a_ref.at[c],
            send_sem=send_sems.at[0],
            recv_sem=recv_sems.at[c],
            device_id=(0,),
            device_id_type=pl.DeviceIdType.MESH,
        ).wait_recv()

    def _dot(cc):
        return jax.lax.dot_general(
            a_ref[cc], w_ref[...].astype(jnp.bfloat16),
            (((1,), (0,)), ((), ())),
            preferred_element_type=jnp.float32,
        )

    @pl.when(c == 0)
    def _():
        acc_ref[n] = _dot(0)

    @pl.when(c == 1)
    def _():
        acc_ref[n] = acc_ref[n] + _dot(1)

    @pl.when(c == 2)
    def _():
        acc_ref[n] = acc_ref[n] + _dot(2)

    @pl.when(c == 3)
    def _():
        final = acc_ref[n] + _dot(3)
        acc_ref[n] = final
        m = jnp.max(jnp.abs(final))

        @pl.when(n == 0)
        def _():
            sm_ref[0] = m

        @pl.when(n > 0)
        def _():
            sm_ref[0] = jnp.maximum(sm_ref[0], m)

    @pl.when((c == 4) & (n == 0))
    def _():
        axc_ref[my] = jnp.full((8, 128), sm_ref[0], jnp.float32)
        for off in range(1, N_DEV):
            d = lax.rem(my + off, N_DEV)
            pltpu.make_async_remote_copy(
                src_ref=axc_ref.at[my],
                dst_ref=axc_ref.at[my],
                send_sem=asend_sems.at[off - 1],
                recv_sem=arecv_sems.at[my],
                device_id=(d,),
                device_id_type=pl.DeviceIdType.MESH,
            ).start()
        for off in range(1, N_DEV):
            d = lax.rem(my + off, N_DEV)
            pltpu.make_async_remote_copy(
                src_ref=axc_ref.at[d],
                dst_ref=axc_ref.at[d],
                send_sem=asend_sems.at[0],
                recv_sem=arecv_sems.at[d],
                device_id=(d,),
                device_id_type=pl.DeviceIdType.MESH,
            ).wait_recv()
        sm_ref[1] = jnp.max(axc_ref[...]) / 448.0

    @pl.when(c == 4)
    def _():
        s = sm_ref[1]
        q = (acc_ref[n] / s).astype(jnp.float8_e4m3fn)
        acc_ref[n] = q.astype(jnp.float32) * s

        @pl.when(n >= 2)
        def _():
            pltpu.make_async_copy(
                acc_ref.at[n - 2],
                out_ref.at[:, pl.ds((n - 2) * BN, BN)],
                out_sems.at[lax.rem(n, 2)],
            ).wait()

        pltpu.make_async_copy(
            acc_ref.at[n],
            out_ref.at[:, pl.ds(n * BN, BN)],
            out_sems.at[lax.rem(n, 2)],
        ).start()

    @pl.when((c == 4) & (n == NB - 1))
    def _():
        pltpu.make_async_copy(
            acc_ref.at[NB - 2], out_ref.at[:, pl.ds((NB - 2) * BN, BN)],
            out_sems.at[0]).wait()
        pltpu.make_async_copy(
            acc_ref.at[NB - 1], out_ref.at[:, pl.ds((NB - 1) * BN, BN)],
            out_sems.at[1]).wait()
        for idx in range(3):
            pltpu.make_async_remote_copy(
                src_ref=xb_ref.at[idx], dst_ref=a_ref.at[1],
                send_sem=send_sems.at[idx], recv_sem=recv_sems.at[1],
                device_id=(0,), device_id_type=pl.DeviceIdType.MESH,
            ).wait_send()
            pltpu.make_async_remote_copy(
                src_ref=axc_ref.at[0], dst_ref=axc_ref.at[0],
                send_sem=asend_sems.at[idx], recv_sem=arecv_sems.at[0],
                device_id=(0,), device_id_type=pl.DeviceIdType.MESH,
            ).wait_send()


def _w_index_map(c, n, perm_ref):
    return (perm_ref[jnp.minimum(c, 3)], jnp.where(c == 4, NB - 1, n))


def kernel(x, w_mat):
    my = lax.axis_index("i")
    perm = jnp.remainder(
        jnp.array([0, 1, 3, 2], jnp.int32) + my.astype(jnp.int32), N_DEV)

    grid_spec = pltpu.PrefetchScalarGridSpec(
        num_scalar_prefetch=1,
        grid=(N_DEV + 1, NB),
        in_specs=[
            pl.BlockSpec(memory_space=pl.ANY),
            pl.BlockSpec((K_BLK, BN), _w_index_map),
        ],
        out_specs=pl.BlockSpec(memory_space=pl.ANY),
        scratch_shapes=[
            pltpu.VMEM((M_LOC, K_BLK), jnp.float32),
            pltpu.VMEM((3, M_LOC, K_BLK), jnp.bfloat16),
            pltpu.VMEM((N_DEV, M_LOC, K_BLK), jnp.bfloat16),
            pltpu.VMEM((NB, M_LOC, BN), jnp.float32),
            pltpu.VMEM((N_DEV, 8, 128), jnp.float32),
            pltpu.SMEM((2,), jnp.float32),
            pltpu.SemaphoreType.DMA,
            pltpu.SemaphoreType.DMA((3,)),
            pltpu.SemaphoreType.DMA((N_DEV,)),
            pltpu.SemaphoreType.DMA((3,)),
            pltpu.SemaphoreType.DMA((N_DEV,)),
            pltpu.SemaphoreType.DMA((2,)),
        ],
    )
    return pl.pallas_call(
        _fused_body,
        grid_spec=grid_spec,
        out_shape=jax.ShapeDtypeStruct((M_LOC, N_OUT), jnp.float32),
        compiler_params=pltpu.CompilerParams(
            dimension_semantics=("arbitrary", "arbitrary"),
            vmem_limit_bytes=64 * 1024 * 1024,
        ),
    )(perm, x, w_mat)


# device time: 144997 ns/iter; 1.1927x vs baseline; 1.1327x over previous
import jax
import jax.numpy as jnp
from jax import lax
from jax.experimental import pallas as pl
from jax.experimental.pallas import tpu as pltpu

N_DEV = 4
M_LOC = 1024
K = 4096
N_OUT = 8192
K_BLK = K // N_DEV
BN = 1024
NB = N_OUT // BN
BQ = 1024

_SEND_OFFS = [3, 1, 2]
_SLOT_FOR_OFF = {3: 1, 1: 2, 2: 3}


def _k1_body(perm_ref, x_ref, w_ref, y_ref, ga_ref,
             chunk_ref, xb_ref, a_ref, acc_ref, axc_ref, sm_ref,
             chunk_sem, send_sems, recv_sems, asend_sems, arecv_sems):
    c = pl.program_id(0)
    n = pl.program_id(1)
    my = lax.axis_index("i")

    half = M_LOC // 2

    def _load_cast(row0, dst_ref, dst0):
        for h in range(2):
            cp = pltpu.make_async_copy(
                x_ref.at[pl.ds(row0 + h * half, half), :],
                chunk_ref, chunk_sem)
            cp.start()
            cp.wait()
            dst_ref[dst0 + h * half:dst0 + (h + 1) * half] = (
                chunk_ref[...].astype(jnp.bfloat16))

    def _prep_send(idx):
        off = _SEND_OFFS[idx]
        d = lax.rem(my + off, N_DEV)
        _load_cast(d * M_LOC, xb_ref.at[idx], 0)
        pltpu.make_async_remote_copy(
            src_ref=xb_ref.at[idx],
            dst_ref=a_ref.at[_SLOT_FOR_OFF[off]],
            send_sem=send_sems.at[idx],
            recv_sem=recv_sems.at[_SLOT_FOR_OFF[off]],
            device_id=(d,),
            device_id_type=pl.DeviceIdType.MESH,
        ).start()

    @pl.when((c == 0) & (n == 0))
    def _():
        _prep_send(0)
        _load_cast(my * M_LOC, a_ref.at[0], 0)

    @pl.when((c == 0) & (n == 1))
    def _():
        _prep_send(1)

    @pl.when((c == 0) & (n == 2))
    def _():
        _prep_send(2)

    @pl.when((c >= 1) & (n == 0))
    def _():
        pltpu.make_async_remote_copy(
            src_ref=xb_ref.at[0],
            dst_ref=a_ref.at[c],
            send_sem=send_sems.at[0],
            recv_sem=recv_sems.at[c],
            device_id=(0,),
            device_id_type=pl.DeviceIdType.MESH,
        ).wait_recv()

    def _dot(cc):
        return jax.lax.dot_general(
            a_ref[cc], w_ref[...].astype(jnp.bfloat16),
            (((1,), (0,)), ((), ())),
            preferred_element_type=jnp.float32,
        )

    @pl.when(c == 0)
    def _():
        acc_ref[n] = _dot(0)

    @pl.when(c == 1)
    def _():
        acc_ref[n] = acc_ref[n] + _dot(1)

    @pl.when(c == 2)
    def _():
        acc_ref[n] = acc_ref[n] + _dot(2)

    @pl.when(c == 3)
    def _():
        final = acc_ref[n] + _dot(3)
        y_ref[...] = final.astype(jnp.bfloat16)
        m = jnp.max(jnp.abs(final))

        @pl.when(n == 0)
        def _():
            sm_ref[0] = m

        @pl.when(n > 0)
        def _():
            sm_ref[0] = jnp.maximum(sm_ref[0], m)

    @pl.when((c == 3) & (n == NB - 1))
    def _():
        axc_ref[my] = jnp.full((8, 128), sm_ref[0], jnp.float32)
        for off in range(1, N_DEV):
            d = lax.rem(my + off, N_DEV)
            pltpu.make_async_remote_copy(
                src_ref=axc_ref.at[my],
                dst_ref=axc_ref.at[my],
                send_sem=asend_sems.at[off - 1],
                recv_sem=arecv_sems.at[my],
                device_id=(d,),
                device_id_type=pl.DeviceIdType.MESH,
            ).start()
        for off in range(1, N_DEV):
            d = lax.rem(my + off, N_DEV)
            pltpu.make_async_remote_copy(
                src_ref=axc_ref.at[d],
                dst_ref=axc_ref.at[d],
                send_sem=asend_sems.at[0],
                recv_sem=arecv_sems.at[d],
                device_id=(d,),
                device_id_type=pl.DeviceIdType.MESH,
            ).wait_recv()
        ga_ref[0, 0] = jnp.max(axc_ref[...])
        for idx in range(3):
            pltpu.make_async_remote_copy(
                src_ref=xb_ref.at[idx], dst_ref=a_ref.at[1],
                send_sem=send_sems.at[idx], recv_sem=recv_sems.at[1],
                device_id=(0,), device_id_type=pl.DeviceIdType.MESH,
            ).wait_send()
            pltpu.make_async_remote_copy(
                src_ref=axc_ref.at[0], dst_ref=axc_ref.at[0],
                send_sem=asend_sems.at[idx], recv_sem=arecv_sems.at[0],
                device_id=(0,), device_id_type=pl.DeviceIdType.MESH,
            ).wait_send()


def _w_index_map(c, n, perm_ref):
    return (perm_ref[c], n)


def _y_index_map(c, n, perm_ref):
    return (0, jnp.where(c == 3, n, 0))


def _k1(perm, x, w_mat):
    grid_spec = pltpu.PrefetchScalarGridSpec(
        num_scalar_prefetch=1,
        grid=(N_DEV, NB),
        in_specs=[
            pl.BlockSpec(memory_space=pl.ANY),
            pl.BlockSpec((K_BLK, BN), _w_index_map),
        ],
        out_specs=[
            pl.BlockSpec((M_LOC, BN), _y_index_map),
            pl.BlockSpec(memory_space=pltpu.SMEM),
        ],
        scratch_shapes=[
            pltpu.VMEM((M_LOC // 2, K_BLK), jnp.float32),
            pltpu.VMEM((3, M_LOC, K_BLK), jnp.bfloat16),
            pltpu.VMEM((N_DEV, M_LOC, K_BLK), jnp.bfloat16),
            pltpu.VMEM((NB, M_LOC, BN), jnp.float32),
            pltpu.VMEM((N_DEV, 8, 128), jnp.float32),
            pltpu.SMEM((1,), jnp.float32),
            pltpu.SemaphoreType.DMA,
            pltpu.SemaphoreType.DMA((3,)),
            pltpu.SemaphoreType.DMA((N_DEV,)),
            pltpu.SemaphoreType.DMA((3,)),
            pltpu.SemaphoreType.DMA((N_DEV,)),
        ],
    )
    return pl.pallas_call(
        _k1_body,
        grid_spec=grid_spec,
        out_shape=[
            jax.ShapeDtypeStruct((M_LOC, N_OUT), jnp.bfloat16),
            jax.ShapeDtypeStruct((1, 1), jnp.float32),
        ],
        compiler_params=pltpu.CompilerParams(
            dimension_semantics=("arbitrary", "arbitrary"),
            vmem_limit_bytes=64 * 1024 * 1024,
        ),
    )(perm, x, w_mat)


def _quant_body(y_ref, ga_ref, out_ref):
    s = ga_ref[0, 0] / 448.0
    q = (y_ref[...].astype(jnp.float32) / s).astype(jnp.float8_e4m3fn)
    out_ref[...] = q.astype(jnp.float32) * s


def _quant(y, ga):
    return pl.pallas_call(
        _quant_body,
        grid=(N_OUT // BQ,),
        in_specs=[
            pl.BlockSpec((M_LOC, BQ), lambda n: (0, n)),
            pl.BlockSpec((1, 1), lambda n: (0, 0), memory_space=pltpu.SMEM),
        ],
        out_specs=pl.BlockSpec((M_LOC, BQ), lambda n: (0, n)),
        out_shape=jax.ShapeDtypeStruct((M_LOC, N_OUT), jnp.float32),
    )(y, ga)


def kernel(x, w_mat):
    my = lax.axis_index("i")
    perm = jnp.remainder(
        jnp.array([0, 1, 3, 2], jnp.int32) + my.astype(jnp.int32), N_DEV)
    y, ga = _k1(perm, x, w_mat)
    return _quant(y, ga)
